# Initial kernel scaffold; baseline (speedup 1.0000x reference)
import jax
import jax.numpy as jnp
from jax.experimental import pallas as pl


def kernel(x, edge_index, lin1_W, lin1_b, gcn_W, gcn_b, sage_Wl, sage_Wr, sage_b, ssum_Wl, ssum_Wr, ssum_b, smax_Wl, smax_Wr, smax_b, gin_W, gin_b, la_W, la_b, cls_W, cls_b):
    raise NotImplementedError("write your pallas kernel here")


# speedup vs baseline: 1.6378x; 1.6378x over previous
import functools

import jax
import jax.numpy as jnp
from jax import lax
from jax.experimental import pallas as pl

N = 10000
E = 320000
H = 128
OUT_DIM = 64
NA_W = 1.0 / (5.0 ** 0.5)
SC_W = 1.0 / (2.0 ** 0.5)
LA_W = 1.0 / (2.0 ** 0.5)

BN = 1000


def _lin1_body(x_ref, w_ref, b_ref, o_ref):
    o_ref[...] = jnp.dot(x_ref[...], w_ref[...],
                         preferred_element_type=jnp.float32) + b_ref[...]


def _tc_lin1(x, w, b):
    return pl.pallas_call(
        _lin1_body,
        grid=(N // BN,),
        in_specs=[
            pl.BlockSpec((BN, H), lambda i: (i, 0)),
            pl.BlockSpec((H, H), lambda i: (0, 0)),
            pl.BlockSpec((1, H), lambda i: (0, 0)),
        ],
        out_specs=pl.BlockSpec((BN, H), lambda i: (i, 0)),
        out_shape=jax.ShapeDtypeStruct((N, H), jnp.float32),
    )(x, w, b.reshape(1, H))


def _mixed_body(x_ref, s_ref, t_ref, m_ref, aux_ref,
                wg_ref, ws_l_ref, ws_r_ref, wu_l_ref, wu_r_ref,
                wm_l_ref, wm_r_ref, wgin_ref, bias_ref, o_ref):
    x = x_ref[...]
    s = s_ref[...]
    dinv = aux_ref[:, 0:1]
    dinv2 = aux_ref[:, 1:2]
    icnt = aux_ref[:, 2:3]
    f32 = jnp.float32
    dot = lambda a, w: jnp.dot(a, w, preferred_element_type=f32)
    u = dinv * t_ref[...] + dinv2 * x
    gcn = dot(u, wg_ref[...]) + bias_ref[0:1, :]
    mean = dot(x, ws_l_ref[...]) + dot(s * icnt, ws_r_ref[...]) + bias_ref[1:2, :]
    ssum = dot(x, wu_l_ref[...]) + dot(s, wu_r_ref[...]) + bias_ref[2:3, :]
    smax = dot(x, wm_l_ref[...]) + dot(m_ref[...], wm_r_ref[...]) + bias_ref[3:4, :]
    gin = dot(x + s, wgin_ref[...]) + bias_ref[4:5, :]
    r = jax.nn.relu
    o_ref[...] = NA_W * (r(gcn) + r(mean) + r(ssum) + r(smax) + r(gin))


def _tc_mixed(x, s, t, m, aux, wg, wsl, wsr, wul, wur, wml, wmr, wgin, bias5):
    wspec = pl.BlockSpec((H, H), lambda i: (0, 0))
    return pl.pallas_call(
        _mixed_body,
        grid=(N // BN,),
        in_specs=[pl.BlockSpec((BN, H), lambda i: (i, 0))] * 4
        + [pl.BlockSpec((BN, 8), lambda i: (i, 0))]
        + [wspec] * 8
        + [pl.BlockSpec((5, H), lambda i: (0, 0))],
        out_specs=pl.BlockSpec((BN, H), lambda i: (i, 0)),
        out_shape=jax.ShapeDtypeStruct((N, H), jnp.float32),
    )(x, s, t, m, aux, wg, wsl, wsr, wul, wur, wml, wmr, wgin, bias5)


def _head_body(x1_ref, x2_ref, x3_ref, la1_ref, la2_ref, la3_ref,
               lab_ref, cw_ref, cb_ref, o_ref):
    x1 = SC_W * x1_ref[...]
    x2 = SC_W * x2_ref[...]
    x3 = x3_ref[...]
    f32 = jnp.float32
    dot = lambda a, w: jnp.dot(a, w, preferred_element_type=f32)
    lmax = jnp.maximum(jnp.maximum(x3, x1), x2)
    lcat = dot(x3, la1_ref[...]) + dot(x1, la2_ref[...]) + dot(x2, la3_ref[...]) + lab_ref[...]
    z = LA_W * (lmax + lcat)
    o_ref[...] = dot(z, cw_ref[...]) + cb_ref[...]


def _tc_head(x1, x2, x3, la_W, la_b, cls_W, cls_b):
    wspec = pl.BlockSpec((H, H), lambda i: (0, 0))
    return pl.pallas_call(
        _head_body,
        grid=(N // BN,),
        in_specs=[pl.BlockSpec((BN, H), lambda i: (i, 0))] * 3
        + [wspec] * 3
        + [pl.BlockSpec((1, H), lambda i: (0, 0)),
           pl.BlockSpec((H, OUT_DIM), lambda i: (0, 0)),
           pl.BlockSpec((1, OUT_DIM), lambda i: (0, 0))],
        out_specs=pl.BlockSpec((BN, OUT_DIM), lambda i: (i, 0)),
        out_shape=jax.ShapeDtypeStruct((N, OUT_DIM), jnp.float32),
    )(x1, x2, x3, la_W[0:H], la_W[H:2 * H], la_W[2 * H:3 * H],
      la_b.reshape(1, H), cls_W, cls_b.reshape(1, OUT_DIM))


def _aggregate(x, s_sorted, d_sorted, dinv_e):
    rows = x[s_sorted]
    S = jax.ops.segment_sum(rows, d_sorted, num_segments=N,
                            indices_are_sorted=True)
    T = jax.ops.segment_sum(rows * dinv_e[:, None], d_sorted, num_segments=N,
                            indices_are_sorted=True)
    M = jax.ops.segment_max(rows, d_sorted, num_segments=N,
                            indices_are_sorted=True)
    return S, T, M


def kernel(x, edge_index, lin1_W, lin1_b, gcn_W, gcn_b, sage_Wl, sage_Wr,
           sage_b, ssum_Wl, ssum_Wr, ssum_b, smax_Wl, smax_Wr, smax_b,
           gin_W, gin_b, la_W, la_b, cls_W, cls_b):
    src = edge_index[0]
    dst = edge_index[1]
    d_sorted, s_sorted = lax.sort((dst, src), num_keys=1)
    offsets = jnp.searchsorted(d_sorted, jnp.arange(N + 1, dtype=jnp.int32))
    cnt = (offsets[1:] - offsets[:-1]).astype(jnp.float32)
    dinv = lax.rsqrt(cnt + 1.0)
    icnt = 1.0 / jnp.maximum(cnt, 1.0)
    has = (cnt > 0.0).astype(jnp.float32)
    aux = jnp.stack([dinv, dinv * dinv, icnt, has,
                     jnp.zeros_like(cnt), jnp.zeros_like(cnt),
                     jnp.zeros_like(cnt), jnp.zeros_like(cnt)], axis=1)
    dinv_e = dinv[s_sorted]

    h = _tc_lin1(x, lin1_W, lin1_b)
    hs = []
    cur = h
    for l in range(3):
        S, T, M = _aggregate(cur, s_sorted, d_sorted, dinv_e)
        M = jnp.where(cnt[:, None] > 0, M, 0.0)
        bias5 = jnp.stack([gcn_b[l], sage_b[l], ssum_b[l], smax_b[l], gin_b[l]])
        cur = _tc_mixed(cur, S, T, M, aux, gcn_W[l], sage_Wl[l], sage_Wr[l],
                        ssum_Wl[l], ssum_Wr[l], smax_Wl[l], smax_Wr[l],
                        gin_W[l], bias5)
        hs.append(cur)
    return _tc_head(hs[0], hs[1], hs[2], la_W, la_b, cls_W, cls_b)


# speedup vs baseline: 3.6559x; 2.2322x over previous
import functools

import jax
import jax.numpy as jnp
from jax import lax
from jax.experimental import pallas as pl
from jax.experimental.pallas import tpu as pltpu
from jax.experimental.pallas import tpu_sc as plsc

N = 10000
E = 320000
H = 128
OUT_DIM = 64
NA_W = 1.0 / (5.0 ** 0.5)
SC_W = 1.0 / (2.0 ** 0.5)
LA_W = 1.0 / (2.0 ** 0.5)

BN = 1000

NT = 32
NPT = 320
N_PAD = NT * NPT
GS = 160
CK = 128
OFFS_STAGE = NPT + 16
E_PAD = E + 2 * CK
NEG = -3.0e38


def _agg_body(x_hbm, src_hbm, dst_hbm, dinve_hbm, offs_hbm,
              outS, outT, outM,
              rows_v, idx_v, dst_v, dinv_v, offs_v, stS, stT, stM,
              semg, sem1, sem2, sem3):
    f32 = jnp.float32
    wid = lax.axis_index("c") * 16 + lax.axis_index("s")
    base_node = wid * NPT
    pltpu.sync_copy(offs_hbm.at[pl.ds(base_node * 1, OFFS_STAGE)], offs_v)

    zero16 = jnp.zeros((16,), f32)
    neg16 = jnp.full((16,), NEG, f32)

    for g in range(2):
        gbase = base_node + g * GS
        eb = offs_v[pl.ds(g * GS, 16)][0]
        ee = offs_v[pl.ds((g + 1) * GS, 16)][0]

        def zero_body(r, _):
            rb = r * H
            for k in range(8):
                stS[pl.ds(rb + 16 * k, 16)] = zero16
                stT[pl.ds(rb + 16 * k, 16)] = zero16
                stM[pl.ds(rb + 16 * k, 16)] = zero16
            return 0
        lax.fori_loop(0, GS, zero_body, 0)

        cb0 = jnp.bitwise_and(eb, jnp.int32(-8))
        nc = (ee - cb0 + (CK - 1)) // CK

        def chunk_body(gc, carry):
            e0 = pl.multiple_of(cb0 + gc * CK, 8)
            c1 = pltpu.async_copy(src_hbm.at[pl.ds(e0, CK)], idx_v, sem1)
            c2 = pltpu.async_copy(dst_hbm.at[pl.ds(e0, CK)],
                                  dst_v.at[pl.ds(0, CK)], sem2)
            c3 = pltpu.async_copy(dinve_hbm.at[pl.ds(e0, CK)],
                                  dinv_v.at[pl.ds(0, CK)], sem3)
            c1.wait()
            c2.wait()
            c3.wait()
            pltpu.async_copy(x_hbm.at[idx_v], rows_v, semg).wait()
            lo = jnp.maximum(eb, e0) - e0
            hi = jnp.minimum(ee, e0 + CK) - e0

            def edge_body(j, c):
                d_prev = c[0]
                d = dst_v[pl.ds(j, 16)][0]
                flush_f = d != d_prev

                @pl.when(jnp.logical_and(flush_f, d_prev >= jnp.int32(0)))
                def _():
                    rb = (d_prev - gbase) * H
                    for k in range(8):
                        stS[pl.ds(rb + 16 * k, 16)] = c[1 + k]
                        stT[pl.ds(rb + 16 * k, 16)] = c[9 + k]
                        stM[pl.ds(rb + 16 * k, 16)] = c[17 + k]

                keep = jnp.where(flush_f, 0.0, 1.0).astype(f32)
                k16 = jnp.full((16,), keep, f32)
                dv = jnp.full((16,), dinv_v[pl.ds(j, 16)][0], f32)
                ns, nt, nm = [], [], []
                for k in range(8):
                    row = rows_v[j, pl.ds(16 * k, 16)]
                    ns.append(c[1 + k] * k16 + row)
                    nt.append(c[9 + k] * k16 + dv * row)
                    nm.append(jnp.maximum(jnp.where(flush_f, neg16, c[17 + k]),
                                          row))
                return (d,) + tuple(ns) + tuple(nt) + tuple(nm)

            return lax.fori_loop(lo, hi, edge_body, carry)

        carry0 = (jnp.int32(-1),) + (zero16,) * 16 + (neg16,) * 8
        carry = lax.fori_loop(0, nc, chunk_body, carry0)

        d_last = carry[0]
        @pl.when(d_last >= jnp.int32(0))
        def _():
            rb = (d_last - gbase) * H
            for k in range(8):
                stS[pl.ds(rb + 16 * k, 16)] = carry[1 + k]
                stT[pl.ds(rb + 16 * k, 16)] = carry[9 + k]
                stM[pl.ds(rb + 16 * k, 16)] = carry[17 + k]

        ob = pl.multiple_of(gbase * H, 128)
        pltpu.sync_copy(stS, outS.at[pl.ds(ob, GS * H)])
        pltpu.sync_copy(stT, outT.at[pl.ds(ob, GS * H)])
        pltpu.sync_copy(stM, outM.at[pl.ds(ob, GS * H)])


_sc_aggregate = pl.kernel(
    _agg_body,
    out_type=[jax.ShapeDtypeStruct((N_PAD * H,), jnp.float32)] * 3,
    mesh=plsc.VectorSubcoreMesh(core_axis_name="c", subcore_axis_name="s"),
    scratch_types=[
        pltpu.VMEM((CK, H), jnp.float32),
        pltpu.VMEM((CK,), jnp.int32),
        pltpu.VMEM((CK + 16,), jnp.int32),
        pltpu.VMEM((CK + 16,), jnp.float32),
        pltpu.VMEM((OFFS_STAGE,), jnp.int32),
        pltpu.VMEM((GS * H,), jnp.float32),
        pltpu.VMEM((GS * H,), jnp.float32),
        pltpu.VMEM((GS * H,), jnp.float32),
        pltpu.SemaphoreType.DMA,
        pltpu.SemaphoreType.DMA,
        pltpu.SemaphoreType.DMA,
        pltpu.SemaphoreType.DMA,
    ],
)


def _lin1_body(x_ref, w_ref, b_ref, o_ref):
    o_ref[...] = jnp.dot(x_ref[...], w_ref[...],
                         preferred_element_type=jnp.float32) + b_ref[...]


def _tc_lin1(x, w, b):
    return pl.pallas_call(
        _lin1_body,
        grid=(N // BN,),
        in_specs=[
            pl.BlockSpec((BN, H), lambda i: (i, 0)),
            pl.BlockSpec((H, H), lambda i: (0, 0)),
            pl.BlockSpec((1, H), lambda i: (0, 0)),
        ],
        out_specs=pl.BlockSpec((BN, H), lambda i: (i, 0)),
        out_shape=jax.ShapeDtypeStruct((N, H), jnp.float32),
    )(x, w, b.reshape(1, H))


def _mixed_body(x_ref, s_ref, t_ref, m_ref, aux_ref,
                wg_ref, ws_l_ref, ws_r_ref, wu_l_ref, wu_r_ref,
                wm_l_ref, wm_r_ref, wgin_ref, bias_ref, o_ref):
    x = x_ref[...]
    s = s_ref[...]
    dinv = aux_ref[:, 0:1]
    dinv2 = aux_ref[:, 1:2]
    icnt = aux_ref[:, 2:3]
    f32 = jnp.float32
    dot = lambda a, w: jnp.dot(a, w, preferred_element_type=f32)
    u = dinv * t_ref[...] + dinv2 * x
    gcn = dot(u, wg_ref[...]) + bias_ref[0:1, :]
    mean = dot(x, ws_l_ref[...]) + dot(s * icnt, ws_r_ref[...]) + bias_ref[1:2, :]
    ssum = dot(x, wu_l_ref[...]) + dot(s, wu_r_ref[...]) + bias_ref[2:3, :]
    smax = dot(x, wm_l_ref[...]) + dot(m_ref[...], wm_r_ref[...]) + bias_ref[3:4, :]
    gin = dot(x + s, wgin_ref[...]) + bias_ref[4:5, :]
    r = jax.nn.relu
    o_ref[...] = NA_W * (r(gcn) + r(mean) + r(ssum) + r(smax) + r(gin))


def _tc_mixed(x, s, t, m, aux, wg, wsl, wsr, wul, wur, wml, wmr, wgin, bias5):
    wspec = pl.BlockSpec((H, H), lambda i: (0, 0))
    return pl.pallas_call(
        _mixed_body,
        grid=(N // BN,),
        in_specs=[pl.BlockSpec((BN, H), lambda i: (i, 0))] * 4
        + [pl.BlockSpec((BN, 8), lambda i: (i, 0))]
        + [wspec] * 8
        + [pl.BlockSpec((5, H), lambda i: (0, 0))],
        out_specs=pl.BlockSpec((BN, H), lambda i: (i, 0)),
        out_shape=jax.ShapeDtypeStruct((N, H), jnp.float32),
    )(x, s, t, m, aux, wg, wsl, wsr, wul, wur, wml, wmr, wgin, bias5)


def _head_body(x1_ref, x2_ref, x3_ref, la1_ref, la2_ref, la3_ref,
               lab_ref, cw_ref, cb_ref, o_ref):
    x1 = SC_W * x1_ref[...]
    x2 = SC_W * x2_ref[...]
    x3 = x3_ref[...]
    f32 = jnp.float32
    dot = lambda a, w: jnp.dot(a, w, preferred_element_type=f32)
    lmax = jnp.maximum(jnp.maximum(x3, x1), x2)
    lcat = dot(x3, la1_ref[...]) + dot(x1, la2_ref[...]) + dot(x2, la3_ref[...]) + lab_ref[...]
    z = LA_W * (lmax + lcat)
    o_ref[...] = dot(z, cw_ref[...]) + cb_ref[...]


def _tc_head(x1, x2, x3, la_W, la_b, cls_W, cls_b):
    wspec = pl.BlockSpec((H, H), lambda i: (0, 0))
    return pl.pallas_call(
        _head_body,
        grid=(N // BN,),
        in_specs=[pl.BlockSpec((BN, H), lambda i: (i, 0))] * 3
        + [wspec] * 3
        + [pl.BlockSpec((1, H), lambda i: (0, 0)),
           pl.BlockSpec((H, OUT_DIM), lambda i: (0, 0)),
           pl.BlockSpec((1, OUT_DIM), lambda i: (0, 0))],
        out_specs=pl.BlockSpec((BN, OUT_DIM), lambda i: (i, 0)),
        out_shape=jax.ShapeDtypeStruct((N, OUT_DIM), jnp.float32),
    )(x1, x2, x3, la_W[0:H], la_W[H:2 * H], la_W[2 * H:3 * H],
      la_b.reshape(1, H), cls_W, cls_b.reshape(1, OUT_DIM))


def _aggregate(x, s_pad, d_pad, dinv_e_pad, offs_pad):
    S, T, M = _sc_aggregate(x, s_pad, d_pad, dinv_e_pad, offs_pad)
    return (S.reshape(N_PAD, H)[:N], T.reshape(N_PAD, H)[:N],
            M.reshape(N_PAD, H)[:N])


def kernel(x, edge_index, lin1_W, lin1_b, gcn_W, gcn_b, sage_Wl, sage_Wr,
           sage_b, ssum_Wl, ssum_Wr, ssum_b, smax_Wl, smax_Wr, smax_b,
           gin_W, gin_b, la_W, la_b, cls_W, cls_b):
    src = edge_index[0]
    dst = edge_index[1]
    d_sorted, s_sorted = lax.sort((dst, src), num_keys=1)
    offsets = jnp.searchsorted(d_sorted, jnp.arange(N + 1, dtype=jnp.int32)
                               ).astype(jnp.int32)
    cnt = (offsets[1:] - offsets[:-1]).astype(jnp.float32)
    dinv = lax.rsqrt(cnt + 1.0)
    icnt = 1.0 / jnp.maximum(cnt, 1.0)
    has = (cnt > 0.0).astype(jnp.float32)
    aux = jnp.stack([dinv, dinv * dinv, icnt, has,
                     jnp.zeros_like(cnt), jnp.zeros_like(cnt),
                     jnp.zeros_like(cnt), jnp.zeros_like(cnt)], axis=1)
    dinv_e = dinv[s_sorted]

    epad = E_PAD - E
    s_pad = jnp.concatenate([s_sorted, jnp.zeros((epad,), jnp.int32)])
    d_pad = jnp.concatenate([d_sorted, jnp.zeros((epad,), jnp.int32)])
    dinv_e_pad = jnp.concatenate([dinv_e, jnp.zeros((epad,), jnp.float32)])
    offs_pad = jnp.concatenate([
        offsets, jnp.full((N_PAD + 16 - N,), E, jnp.int32)])

    h = _tc_lin1(x, lin1_W, lin1_b)
    hs = []
    cur = h
    for l in range(3):
        S, T, M = _aggregate(cur, s_pad, d_pad, dinv_e_pad, offs_pad)
        bias5 = jnp.stack([gcn_b[l], sage_b[l], ssum_b[l], smax_b[l], gin_b[l]])
        cur = _tc_mixed(cur, S, T, M, aux, gcn_W[l], sage_Wl[l], sage_Wr[l],
                        ssum_Wl[l], ssum_Wr[l], smax_Wl[l], smax_Wr[l],
                        gin_W[l], bias5)
        hs.append(cur)
    return _tc_head(hs[0], hs[1], hs[2], la_W, la_b, cls_W, cls_b)


# speedup vs baseline: 3.6751x; 1.0052x over previous
import functools

import jax
import jax.numpy as jnp
from jax import lax
from jax.experimental import pallas as pl
from jax.experimental.pallas import tpu as pltpu
from jax.experimental.pallas import tpu_sc as plsc

N = 10000
E = 320000
H = 128
OUT_DIM = 64
NA_W = 1.0 / (5.0 ** 0.5)
SC_W = 1.0 / (2.0 ** 0.5)
LA_W = 1.0 / (2.0 ** 0.5)

BN = 1000

NT = 32
NPT = 320
N_PAD = NT * NPT
GS = 160
CK = 128
OFFS_STAGE = NPT + 16
E_PAD = E + 2 * CK
NEG = -3.0e38


def _agg_body(x_hbm, src_hbm, dst_hbm, dinve_hbm, offs_hbm,
              outS, outT, outM,
              rows_v, idx_v, dst_v, dinv_v, offs_v, stS, stT, stM,
              semg, sem1, sem2, sem3):
    f32 = jnp.float32
    wid = lax.axis_index("c") * 16 + lax.axis_index("s")
    base_node = wid * NPT
    pltpu.sync_copy(offs_hbm.at[pl.ds(base_node * 1, OFFS_STAGE)], offs_v)

    zero16 = jnp.zeros((16,), f32)
    neg16 = jnp.full((16,), NEG, f32)

    for g in range(2):
        gbase = base_node + g * GS
        eb = offs_v[pl.ds(g * GS, 16)][0]
        ee = offs_v[pl.ds((g + 1) * GS, 16)][0]

        def zero_body(r, _):
            rb = r * H
            for k in range(8):
                stS[pl.ds(rb + 16 * k, 16)] = zero16
                stT[pl.ds(rb + 16 * k, 16)] = zero16
                stM[pl.ds(rb + 16 * k, 16)] = zero16
            return 0
        lax.fori_loop(0, GS, zero_body, 0)

        cb0 = jnp.bitwise_and(eb, jnp.int32(-8))
        nc = (ee - cb0 + (CK - 1)) // CK

        def chunk_body(gc, carry):
            e0 = pl.multiple_of(cb0 + gc * CK, 8)
            c1 = pltpu.async_copy(src_hbm.at[pl.ds(e0, CK)], idx_v, sem1)
            c2 = pltpu.async_copy(dst_hbm.at[pl.ds(e0, CK)],
                                  dst_v.at[pl.ds(0, CK)], sem2)
            c3 = pltpu.async_copy(dinve_hbm.at[pl.ds(e0, CK)],
                                  dinv_v.at[pl.ds(0, CK)], sem3)
            c1.wait()
            c2.wait()
            c3.wait()
            pltpu.async_copy(x_hbm.at[idx_v], rows_v, semg).wait()
            lo = jnp.maximum(eb, e0) - e0
            hi = jnp.minimum(ee, e0 + CK) - e0

            def edge_body(j, c):
                d_prev = c[0]
                d = dst_v[pl.ds(j, 16)][0]
                flush_f = d != d_prev

                @pl.when(jnp.logical_and(flush_f, d_prev >= jnp.int32(0)))
                def _():
                    rb = (d_prev - gbase) * H
                    for k in range(8):
                        stS[pl.ds(rb + 16 * k, 16)] = c[1 + k]
                        stT[pl.ds(rb + 16 * k, 16)] = c[9 + k]
                        stM[pl.ds(rb + 16 * k, 16)] = c[17 + k]

                keep = jnp.where(flush_f, 0.0, 1.0).astype(f32)
                k16 = jnp.full((16,), keep, f32)
                dv = jnp.full((16,), dinv_v[pl.ds(j, 16)][0], f32)
                ns, nt, nm = [], [], []
                for k in range(8):
                    row = rows_v[j, pl.ds(16 * k, 16)]
                    ns.append(c[1 + k] * k16 + row)
                    nt.append(c[9 + k] * k16 + dv * row)
                    nm.append(jnp.maximum(jnp.where(flush_f, neg16, c[17 + k]),
                                          row))
                return (d,) + tuple(ns) + tuple(nt) + tuple(nm)

            return lax.fori_loop(lo, hi, edge_body, carry)

        carry0 = (jnp.int32(-1),) + (zero16,) * 16 + (neg16,) * 8
        carry = lax.fori_loop(0, nc, chunk_body, carry0)

        d_last = carry[0]
        @pl.when(d_last >= jnp.int32(0))
        def _():
            rb = (d_last - gbase) * H
            for k in range(8):
                stS[pl.ds(rb + 16 * k, 16)] = carry[1 + k]
                stT[pl.ds(rb + 16 * k, 16)] = carry[9 + k]
                stM[pl.ds(rb + 16 * k, 16)] = carry[17 + k]

        ob = pl.multiple_of(gbase * H, 128)
        pltpu.sync_copy(stS, outS.at[pl.ds(ob, GS * H)])
        pltpu.sync_copy(stT, outT.at[pl.ds(ob, GS * H)])
        pltpu.sync_copy(stM, outM.at[pl.ds(ob, GS * H)])


_sc_aggregate = pl.kernel(
    _agg_body,
    out_type=[jax.ShapeDtypeStruct((N_PAD * H,), jnp.float32)] * 3,
    mesh=plsc.VectorSubcoreMesh(core_axis_name="c", subcore_axis_name="s"),
    scratch_types=[
        pltpu.VMEM((CK, H), jnp.float32),
        pltpu.VMEM((CK,), jnp.int32),
        pltpu.VMEM((CK + 16,), jnp.int32),
        pltpu.VMEM((CK + 16,), jnp.float32),
        pltpu.VMEM((OFFS_STAGE,), jnp.int32),
        pltpu.VMEM((GS * H,), jnp.float32),
        pltpu.VMEM((GS * H,), jnp.float32),
        pltpu.VMEM((GS * H,), jnp.float32),
        pltpu.SemaphoreType.DMA,
        pltpu.SemaphoreType.DMA,
        pltpu.SemaphoreType.DMA,
        pltpu.SemaphoreType.DMA,
    ],
)


def _lin1_body(x_ref, w_ref, b_ref, o_ref):
    o_ref[...] = jnp.dot(x_ref[...], w_ref[...],
                         preferred_element_type=jnp.float32) + b_ref[...]


def _tc_lin1(x, w, b):
    return pl.pallas_call(
        _lin1_body,
        grid=(N // BN,),
        in_specs=[
            pl.BlockSpec((BN, H), lambda i: (i, 0)),
            pl.BlockSpec((H, H), lambda i: (0, 0)),
            pl.BlockSpec((1, H), lambda i: (0, 0)),
        ],
        out_specs=pl.BlockSpec((BN, H), lambda i: (i, 0)),
        out_shape=jax.ShapeDtypeStruct((N, H), jnp.float32),
    )(x, w, b.reshape(1, H))


def _mixed_body(x_ref, s_ref, t_ref, m_ref, aux_ref,
                wg_ref, ws_l_ref, ws_r_ref, wu_l_ref, wu_r_ref,
                wm_l_ref, wm_r_ref, wgin_ref, bias_ref, o_ref):
    x = x_ref[...]
    s = s_ref[...]
    dinv = aux_ref[:, 0:1]
    dinv2 = aux_ref[:, 1:2]
    icnt = aux_ref[:, 2:3]
    f32 = jnp.float32
    dot = lambda a, w: jnp.dot(a, w, preferred_element_type=f32)
    u = dinv * t_ref[...] + dinv2 * x
    gcn = dot(u, wg_ref[...]) + bias_ref[0:1, :]
    mean = dot(x, ws_l_ref[...]) + dot(s * icnt, ws_r_ref[...]) + bias_ref[1:2, :]
    ssum = dot(x, wu_l_ref[...]) + dot(s, wu_r_ref[...]) + bias_ref[2:3, :]
    smax = dot(x, wm_l_ref[...]) + dot(m_ref[...], wm_r_ref[...]) + bias_ref[3:4, :]
    gin = dot(x + s, wgin_ref[...]) + bias_ref[4:5, :]
    r = jax.nn.relu
    o_ref[...] = NA_W * (r(gcn) + r(mean) + r(ssum) + r(smax) + r(gin))


def _tc_mixed(x, s, t, m, aux, wg, wsl, wsr, wul, wur, wml, wmr, wgin, bias5):
    wspec = pl.BlockSpec((H, H), lambda i: (0, 0))
    return pl.pallas_call(
        _mixed_body,
        grid=(N // BN,),
        in_specs=[pl.BlockSpec((BN, H), lambda i: (i, 0))] * 4
        + [pl.BlockSpec((BN, 8), lambda i: (i, 0))]
        + [wspec] * 8
        + [pl.BlockSpec((5, H), lambda i: (0, 0))],
        out_specs=pl.BlockSpec((BN, H), lambda i: (i, 0)),
        out_shape=jax.ShapeDtypeStruct((N, H), jnp.float32),
    )(x, s, t, m, aux, wg, wsl, wsr, wul, wur, wml, wmr, wgin, bias5)


def _head_body(x1_ref, x2_ref, x3_ref, la1_ref, la2_ref, la3_ref,
               lab_ref, cw_ref, cb_ref, o_ref):
    x1 = SC_W * x1_ref[...]
    x2 = SC_W * x2_ref[...]
    x3 = x3_ref[...]
    f32 = jnp.float32
    dot = lambda a, w: jnp.dot(a, w, preferred_element_type=f32)
    lmax = jnp.maximum(jnp.maximum(x3, x1), x2)
    lcat = dot(x3, la1_ref[...]) + dot(x1, la2_ref[...]) + dot(x2, la3_ref[...]) + lab_ref[...]
    z = LA_W * (lmax + lcat)
    o_ref[...] = dot(z, cw_ref[...]) + cb_ref[...]


def _tc_head(x1, x2, x3, la_W, la_b, cls_W, cls_b):
    wspec = pl.BlockSpec((H, H), lambda i: (0, 0))
    return pl.pallas_call(
        _head_body,
        grid=(N // BN,),
        in_specs=[pl.BlockSpec((BN, H), lambda i: (i, 0))] * 3
        + [wspec] * 3
        + [pl.BlockSpec((1, H), lambda i: (0, 0)),
           pl.BlockSpec((H, OUT_DIM), lambda i: (0, 0)),
           pl.BlockSpec((1, OUT_DIM), lambda i: (0, 0))],
        out_specs=pl.BlockSpec((BN, OUT_DIM), lambda i: (i, 0)),
        out_shape=jax.ShapeDtypeStruct((N, OUT_DIM), jnp.float32),
    )(x1, x2, x3, la_W[0:H], la_W[H:2 * H], la_W[2 * H:3 * H],
      la_b.reshape(1, H), cls_W, cls_b.reshape(1, OUT_DIM))


def _aggregate(x, s_pad, d_pad, dinv_e_pad, offs_pad):
    S, T, M = _sc_aggregate(x, s_pad, d_pad, dinv_e_pad, offs_pad)
    return (S.reshape(N_PAD, H)[:N], T.reshape(N_PAD, H)[:N],
            M.reshape(N_PAD, H)[:N])


def kernel(x, edge_index, lin1_W, lin1_b, gcn_W, gcn_b, sage_Wl, sage_Wr,
           sage_b, ssum_Wl, ssum_Wr, ssum_b, smax_Wl, smax_Wr, smax_b,
           gin_W, gin_b, la_W, la_b, cls_W, cls_b):
    src = edge_index[0]
    dst = edge_index[1]
    key = lax.sort(dst * 16384 + src)
    d_sorted = key >> 14
    s_sorted = key & 16383
    offsets = jnp.searchsorted(key, jnp.arange(N + 1, dtype=jnp.int32) * 16384
                               ).astype(jnp.int32)
    cnt = (offsets[1:] - offsets[:-1]).astype(jnp.float32)
    dinv = lax.rsqrt(cnt + 1.0)
    icnt = 1.0 / jnp.maximum(cnt, 1.0)
    has = (cnt > 0.0).astype(jnp.float32)
    aux = jnp.stack([dinv, dinv * dinv, icnt, has,
                     jnp.zeros_like(cnt), jnp.zeros_like(cnt),
                     jnp.zeros_like(cnt), jnp.zeros_like(cnt)], axis=1)
    dinv_e = dinv[s_sorted]

    epad = E_PAD - E
    s_pad = jnp.concatenate([s_sorted, jnp.zeros((epad,), jnp.int32)])
    d_pad = jnp.concatenate([d_sorted, jnp.zeros((epad,), jnp.int32)])
    dinv_e_pad = jnp.concatenate([dinv_e, jnp.zeros((epad,), jnp.float32)])
    offs_pad = jnp.concatenate([
        offsets, jnp.full((N_PAD + 16 - N,), E, jnp.int32)])

    h = _tc_lin1(x, lin1_W, lin1_b)
    hs = []
    cur = h
    for l in range(3):
        S, T, M = _aggregate(cur, s_pad, d_pad, dinv_e_pad, offs_pad)
        bias5 = jnp.stack([gcn_b[l], sage_b[l], ssum_b[l], smax_b[l], gin_b[l]])
        cur = _tc_mixed(cur, S, T, M, aux, gcn_W[l], sage_Wl[l], sage_Wr[l],
                        ssum_Wl[l], ssum_Wr[l], smax_Wl[l], smax_Wr[l],
                        gin_W[l], bias5)
        hs.append(cur)
    return _tc_head(hs[0], hs[1], hs[2], la_W, la_b, cls_W, cls_b)
